# baseline (speedup 1.0000x reference)
import jax
import jax.numpy as jnp
from jax.experimental import pallas as pl

_N_E = 1024
_E_DIM = 256
_NUM_Q = 4
_BETA = 0.25
_ROWS_PER_BLOCK = 512


def _rvq_block_kernel(x_ref, cb_ref, cbn_ref, xq_ref, idx_ref, rloss_ref):
    res = x_ref[...]
    xq_acc = jnp.zeros_like(res)
    idx_cols = []
    rloss_cols = []
    iota = jax.lax.broadcasted_iota(jnp.int32, (res.shape[0], _N_E), 1)
    for qi in range(_NUM_Q):
        cb = cb_ref[qi]
        cbn = cbn_ref[qi]
        zn = jnp.sum(res * res, axis=1, keepdims=True)
        s = jax.lax.dot_general(
            res, cb, (((1,), (1,)), ((), ())),
            preferred_element_type=jnp.float32)
        d = (zn + cbn) - 2.0 * s
        dmin = jnp.min(d, axis=1, keepdims=True)
        idx = jnp.min(jnp.where(d == dmin, iota, _N_E), axis=1, keepdims=True)
        onehot = (iota == idx).astype(jnp.float32)
        xq = jax.lax.dot_general(
            onehot, cb, (((1,), (0,)), ((), ())),
            preferred_element_type=jnp.float32,
            precision=jax.lax.Precision.HIGHEST)
        delta = xq - res
        rloss_cols.append(jnp.sum(delta * delta, axis=1, keepdims=True))
        xq_st = res + delta
        res = res - xq_st
        xq_acc = xq_acc + xq_st
        idx_cols.append(idx)
    xq_ref[...] = xq_acc
    idx_ref[...] = jnp.concatenate(idx_cols, axis=1)
    rloss_ref[...] = jnp.concatenate(rloss_cols, axis=1)


def kernel(x, codebooks):
    b, t, e = x.shape
    n_rows = b * t
    x2d = x.reshape(n_rows, e)
    cbn = jnp.sum(codebooks * codebooks, axis=2).reshape(_NUM_Q, 1, _N_E)
    grid = (n_rows // _ROWS_PER_BLOCK,)
    xq2d, idx2d, rloss = pl.pallas_call(
        _rvq_block_kernel,
        grid=grid,
        in_specs=[
            pl.BlockSpec((_ROWS_PER_BLOCK, e), lambda i: (i, 0)),
            pl.BlockSpec((_NUM_Q, _N_E, e), lambda i: (0, 0, 0)),
            pl.BlockSpec((_NUM_Q, 1, _N_E), lambda i: (0, 0, 0)),
        ],
        out_specs=[
            pl.BlockSpec((_ROWS_PER_BLOCK, e), lambda i: (i, 0)),
            pl.BlockSpec((_ROWS_PER_BLOCK, _NUM_Q), lambda i: (i, 0)),
            pl.BlockSpec((_ROWS_PER_BLOCK, _NUM_Q), lambda i: (i, 0)),
        ],
        out_shape=[
            jax.ShapeDtypeStruct((n_rows, e), jnp.float32),
            jax.ShapeDtypeStruct((n_rows, _NUM_Q), jnp.int32),
            jax.ShapeDtypeStruct((n_rows, _NUM_Q), jnp.float32),
        ],
    )(x2d, codebooks, cbn)
    x_q = xq2d.reshape(b, t, e)
    all_indices = idx2d.reshape(b, t, _NUM_Q)
    m = jnp.sum(rloss, axis=0) / (n_rows * e)
    losses = m + _BETA * m
    mean_losses = jnp.mean(losses)
    return (x_q, mean_losses, all_indices)


# speedup vs baseline: 1.9030x; 1.9030x over previous
import jax
import jax.numpy as jnp
from jax.experimental import pallas as pl

_N_E = 1024
_E_DIM = 256
_NUM_Q = 4
_BETA = 0.25
_ROWS_PER_BLOCK = 512


def _rvq_block_kernel(x_ref, cb_ref, cbhi_ref, cbmid_ref, cbn_ref,
                      xq_ref, idx_ref, rloss_ref):
    res = x_ref[...]
    xq_acc = jnp.zeros_like(res)
    idx_cols = []
    rloss_cols = []
    iota = jax.lax.broadcasted_iota(jnp.int32, (res.shape[0], _N_E), 1)
    for qi in range(_NUM_Q):
        cb = cb_ref[qi]
        cbn = cbn_ref[qi]
        zn = jnp.sum(res * res, axis=1, keepdims=True)
        s = jax.lax.dot_general(
            res, cb, (((1,), (1,)), ((), ())),
            preferred_element_type=jnp.float32)
        d = (zn + cbn) - 2.0 * s
        dmin = jnp.min(d, axis=1, keepdims=True)
        idx = jnp.min(jnp.where(d == dmin, iota, _N_E), axis=1, keepdims=True)
        onehot = (iota == idx).astype(jnp.bfloat16)
        xq_hi = jax.lax.dot_general(
            onehot, cbhi_ref[qi], (((1,), (0,)), ((), ())),
            preferred_element_type=jnp.float32)
        xq_mid = jax.lax.dot_general(
            onehot, cbmid_ref[qi], (((1,), (0,)), ((), ())),
            preferred_element_type=jnp.float32)
        xq = xq_hi + xq_mid
        delta = xq - res
        rloss_cols.append(jnp.sum(delta * delta, axis=1, keepdims=True))
        xq_st = res + delta
        res = res - xq_st
        xq_acc = xq_acc + xq_st
        idx_cols.append(idx)
    xq_ref[...] = xq_acc
    idx_ref[...] = jnp.concatenate(idx_cols, axis=1)
    rloss_ref[...] = jnp.concatenate(rloss_cols, axis=1)


def kernel(x, codebooks):
    b, t, e = x.shape
    n_rows = b * t
    x2d = x.reshape(n_rows, e)
    cbn = jnp.sum(codebooks * codebooks, axis=2).reshape(_NUM_Q, 1, _N_E)
    cb_hi = codebooks.astype(jnp.bfloat16)
    cb_mid = (codebooks - cb_hi.astype(jnp.float32)).astype(jnp.bfloat16)
    grid = (n_rows // _ROWS_PER_BLOCK,)
    xq2d, idx2d, rloss = pl.pallas_call(
        _rvq_block_kernel,
        grid=grid,
        in_specs=[
            pl.BlockSpec((_ROWS_PER_BLOCK, e), lambda i: (i, 0)),
            pl.BlockSpec((_NUM_Q, _N_E, e), lambda i: (0, 0, 0)),
            pl.BlockSpec((_NUM_Q, _N_E, e), lambda i: (0, 0, 0)),
            pl.BlockSpec((_NUM_Q, _N_E, e), lambda i: (0, 0, 0)),
            pl.BlockSpec((_NUM_Q, 1, _N_E), lambda i: (0, 0, 0)),
        ],
        out_specs=[
            pl.BlockSpec((_ROWS_PER_BLOCK, e), lambda i: (i, 0)),
            pl.BlockSpec((_ROWS_PER_BLOCK, _NUM_Q), lambda i: (i, 0)),
            pl.BlockSpec((_ROWS_PER_BLOCK, _NUM_Q), lambda i: (i, 0)),
        ],
        out_shape=[
            jax.ShapeDtypeStruct((n_rows, e), jnp.float32),
            jax.ShapeDtypeStruct((n_rows, _NUM_Q), jnp.int32),
            jax.ShapeDtypeStruct((n_rows, _NUM_Q), jnp.float32),
        ],
    )(x2d, codebooks, cb_hi, cb_mid, cbn)
    x_q = xq2d.reshape(b, t, e)
    all_indices = idx2d.reshape(b, t, _NUM_Q)
    m = jnp.sum(rloss, axis=0) / (n_rows * e)
    losses = m + _BETA * m
    mean_losses = jnp.mean(losses)
    return (x_q, mean_losses, all_indices)
